# baseline (speedup 1.0000x reference)
import jax
import jax.numpy as jnp
from jax.experimental import pallas as pl
from jax.experimental.pallas import tpu as pltpu

_N = 1_000_000
_BIG = 166_656
_PER_CORE = 3
_TAIL_OFF = 2 * _PER_CORE * _BIG
_TAIL = _N - _TAIL_OFF


def _copy_body(in_hbm, out_hbm, buf0, buf1, buf2, tailbuf, in_sem, out_sem):
    pid = pl.program_id(0)
    bufs = (buf0, buf1, buf2)
    offs = [
        pl.multiple_of(pid * (_PER_CORE * _BIG) + j * _BIG, 128)
        for j in range(_PER_CORE)
    ]
    for j in range(_PER_CORE):
        pltpu.make_async_copy(
            in_hbm.at[pl.ds(offs[j], _BIG)], bufs[j], in_sem.at[j]
        ).start()

    @pl.when(pid == 1)
    def _():
        pltpu.make_async_copy(
            in_hbm.at[pl.ds(_TAIL_OFF, _TAIL)], tailbuf, in_sem.at[_PER_CORE]
        ).start()

    for j in range(_PER_CORE):
        pltpu.make_async_copy(
            in_hbm.at[pl.ds(offs[j], _BIG)], bufs[j], in_sem.at[j]
        ).wait()
        pltpu.make_async_copy(
            bufs[j], out_hbm.at[pl.ds(offs[j], _BIG)], out_sem.at[j]
        ).start()

    @pl.when(pid == 1)
    def _():
        pltpu.make_async_copy(
            in_hbm.at[pl.ds(_TAIL_OFF, _TAIL)], tailbuf, in_sem.at[_PER_CORE]
        ).wait()
        pltpu.make_async_copy(
            tailbuf, out_hbm.at[pl.ds(_TAIL_OFF, _TAIL)], out_sem.at[_PER_CORE]
        ).start()

    for j in range(_PER_CORE):
        pltpu.make_async_copy(
            bufs[j], out_hbm.at[pl.ds(offs[j], _BIG)], out_sem.at[j]
        ).wait()

    @pl.when(pid == 1)
    def _():
        pltpu.make_async_copy(
            tailbuf, out_hbm.at[pl.ds(_TAIL_OFF, _TAIL)], out_sem.at[_PER_CORE]
        ).wait()


def kernel(goal_logits):
    return pl.pallas_call(
        _copy_body,
        out_shape=jax.ShapeDtypeStruct((_N,), jnp.float32),
        grid=(2,),
        in_specs=[pl.BlockSpec(memory_space=pl.ANY)],
        out_specs=pl.BlockSpec(memory_space=pl.ANY),
        scratch_shapes=[
            pltpu.VMEM((_BIG,), jnp.float32),
            pltpu.VMEM((_BIG,), jnp.float32),
            pltpu.VMEM((_BIG,), jnp.float32),
            pltpu.VMEM((_TAIL,), jnp.float32),
            pltpu.SemaphoreType.DMA((_PER_CORE + 1,)),
            pltpu.SemaphoreType.DMA((_PER_CORE + 1,)),
        ],
        compiler_params=pltpu.CompilerParams(
            dimension_semantics=("parallel",),
        ),
    )(goal_logits)


# speedup vs baseline: 1.4803x; 1.4803x over previous
import jax
import jax.numpy as jnp
from jax.experimental import pallas as pl
from jax.experimental.pallas import tpu as pltpu

_N = 1_000_000
_NCHUNK = 9
_BIG = 999_936 // _NCHUNK
_TAIL_OFF = _NCHUNK * _BIG
_TAIL = _N - _TAIL_OFF
_OFFS = tuple(i * _BIG for i in range(_NCHUNK)) + (_TAIL_OFF,)
_SIZES = (_BIG,) * _NCHUNK + (_TAIL,)
_NPIECE = _NCHUNK + 1


def _copy_body(in_hbm, out_hbm, *rest):
    bufs = rest[:_NPIECE]
    in_sem, out_sem = rest[_NPIECE], rest[_NPIECE + 1]
    for i in range(_NPIECE):
        pltpu.make_async_copy(
            in_hbm.at[pl.ds(_OFFS[i], _SIZES[i])], bufs[i], in_sem.at[i]
        ).start()
    for i in range(_NPIECE):
        pltpu.make_async_copy(
            in_hbm.at[pl.ds(_OFFS[i], _SIZES[i])], bufs[i], in_sem.at[i]
        ).wait()
        pltpu.make_async_copy(
            bufs[i], out_hbm.at[pl.ds(_OFFS[i], _SIZES[i])], out_sem.at[i]
        ).start()
    for i in range(_NPIECE):
        pltpu.make_async_copy(
            bufs[i], out_hbm.at[pl.ds(_OFFS[i], _SIZES[i])], out_sem.at[i]
        ).wait()


def kernel(goal_logits):
    return pl.pallas_call(
        _copy_body,
        out_shape=jax.ShapeDtypeStruct((_N,), jnp.float32),
        in_specs=[pl.BlockSpec(memory_space=pl.ANY)],
        out_specs=pl.BlockSpec(memory_space=pl.ANY),
        scratch_shapes=(
            [pltpu.VMEM((s,), jnp.float32) for s in _SIZES]
            + [pltpu.SemaphoreType.DMA((_NPIECE,)),
               pltpu.SemaphoreType.DMA((_NPIECE,))]
        ),
    )(goal_logits)


# speedup vs baseline: 1.4979x; 1.0119x over previous
import jax
import jax.numpy as jnp
from jax.experimental import pallas as pl
from jax.experimental.pallas import tpu as pltpu

_N = 1_000_000
_NCHUNK = 6
_BIG = 999_936 // _NCHUNK
_TAIL_OFF = _NCHUNK * _BIG
_TAIL = _N - _TAIL_OFF
_OFFS = tuple(i * _BIG for i in range(_NCHUNK)) + (_TAIL_OFF,)
_SIZES = (_BIG,) * _NCHUNK + (_TAIL,)
_NPIECE = _NCHUNK + 1


def _copy_body(in_hbm, out_hbm, *rest):
    bufs = rest[:_NPIECE]
    in_sem, out_sem = rest[_NPIECE], rest[_NPIECE + 1]
    for i in range(_NPIECE):
        pltpu.make_async_copy(
            in_hbm.at[pl.ds(_OFFS[i], _SIZES[i])], bufs[i], in_sem.at[i]
        ).start()
    for i in range(_NPIECE):
        pltpu.make_async_copy(
            in_hbm.at[pl.ds(_OFFS[i], _SIZES[i])], bufs[i], in_sem.at[i]
        ).wait()
        pltpu.make_async_copy(
            bufs[i], out_hbm.at[pl.ds(_OFFS[i], _SIZES[i])], out_sem.at[i]
        ).start()
    for i in range(_NPIECE):
        pltpu.make_async_copy(
            bufs[i], out_hbm.at[pl.ds(_OFFS[i], _SIZES[i])], out_sem.at[i]
        ).wait()


def kernel(goal_logits):
    return pl.pallas_call(
        _copy_body,
        out_shape=jax.ShapeDtypeStruct((_N,), jnp.float32),
        in_specs=[pl.BlockSpec(memory_space=pl.ANY)],
        out_specs=pl.BlockSpec(memory_space=pl.ANY),
        scratch_shapes=(
            [pltpu.VMEM((s,), jnp.float32) for s in _SIZES]
            + [pltpu.SemaphoreType.DMA((_NPIECE,)),
               pltpu.SemaphoreType.DMA((_NPIECE,))]
        ),
    )(goal_logits)
